# baseline (speedup 1.0000x reference)
import functools

import jax
import jax.numpy as jnp
from jax import lax
from jax.experimental import pallas as pl

_NUM_CLASSES = 81
_B, _N = 32, 24564
_R = _B * _N
_ROWS = 8832
_STEPS = _R // _ROWS


def _ssd_loss_kernel(conf_ref, tgt_ref, locp_ref, loct_ref, out_ref):
    step = pl.program_id(0)

    x = conf_ref[...]
    tgt = tgt_ref[...]
    pos = (tgt > 0).astype(jnp.float32)

    m = jnp.max(x, axis=1, keepdims=True)
    s = jnp.sum(jnp.exp(x - m), axis=1, keepdims=True)
    lse = m + jnp.log(s)

    cls = lax.broadcasted_iota(jnp.int32, (_ROWS, _NUM_CLASSES), 1)
    tgt_logit = jnp.sum(jnp.where(cls == tgt, x, 0.0), axis=1, keepdims=True)

    conf_part = jnp.sum(pos * (lse - tgt_logit))

    d = locp_ref[...] - loct_ref[...]
    ad = jnp.abs(d)
    elem = jnp.where(ad < 1.0, 0.5 * d * d, ad - 0.5)
    loc_part = jnp.sum(jnp.sum(elem, axis=1, keepdims=True) * pos)

    npos_part = jnp.sum(pos)

    lane = lax.broadcasted_iota(jnp.int32, (1, 128), 1)
    vec = (jnp.where(lane == 0, conf_part, 0.0)
           + jnp.where(lane == 1, loc_part, 0.0)
           + jnp.where(lane == 2, npos_part, 0.0))

    @pl.when(step == 0)
    def _init():
        out_ref[...] = jnp.zeros_like(out_ref)

    out_ref[...] += vec


@jax.jit
def kernel(loc_pred, conf_pred, loc_target, conf_target, default_boxes):
    conf = conf_pred.reshape(_R, _NUM_CLASSES)
    tgt = conf_target.reshape(_R, 1)
    locp = loc_pred.reshape(_R, 4)
    loct = loc_target.reshape(_R, 4)

    out = pl.pallas_call(
        _ssd_loss_kernel,
        grid=(_STEPS,),
        in_specs=[
            pl.BlockSpec((_ROWS, _NUM_CLASSES), lambda i: (i, 0)),
            pl.BlockSpec((_ROWS, 1), lambda i: (i, 0)),
            pl.BlockSpec((_ROWS, 4), lambda i: (i, 0)),
            pl.BlockSpec((_ROWS, 4), lambda i: (i, 0)),
        ],
        out_specs=pl.BlockSpec((1, 128), lambda i: (0, 0)),
        out_shape=jax.ShapeDtypeStruct((1, 128), jnp.float32),
    )(conf, tgt, locp, loct)

    conf_sum = out[0, 0]
    loc_sum = out[0, 1]
    num_pos = out[0, 2]

    conf_loss = jnp.where(num_pos > 0, conf_sum / jnp.maximum(num_pos, 1.0), 0.0)
    loc_loss = jnp.where(num_pos > 0, loc_sum / jnp.maximum(num_pos * 4.0, 1.0), 0.0)
    total_loss = conf_loss + loc_loss
    return (total_loss, conf_loss, loc_loss)
